# baseline (speedup 1.0000x reference)
import functools

import jax
import jax.numpy as jnp
from jax import lax
from jax.experimental import pallas as pl
from jax.experimental.pallas import tpu as pltpu
from jax.experimental.pallas import tpu_sc as plsc

B = 4096
L = 50
D = 128
NC = 2
NS = 16
NW = NC * NS
SPW = B // NW
IPW = SPW * L
NB = 12
DV = D // 16
_SHIFT = [2 * (b % 4) for b in range(NB)]
_GLEN = [L + sh for sh in _SHIFT]
_FULL = (SPW // NB) * NB
_TAIL = SPW - _FULL


def _sc_segment_sums(data, table):
    mesh = plsc.VectorSubcoreMesh(core_axis_name="c", subcore_axis_name="s")

    @functools.partial(
        pl.kernel,
        mesh=mesh,
        out_type=jax.ShapeDtypeStruct((B, D), jnp.float32),
        scratch_types=[
            pltpu.VMEM((IPW,), jnp.int32),
        ] + [
            pltpu.VMEM((_GLEN[b], D), jnp.float32) for b in range(NB)
        ] + [
            pltpu.VMEM((SPW, D), jnp.float32),
        ] + [pltpu.SemaphoreType.DMA] * NB,
    )
    def k(data_hbm, table_hbm, out_hbm, idx_v, *sc):
        rows_bufs, acc, sems = sc[:NB], sc[NB], sc[NB + 1:]
        wid = lax.axis_index("s") * NC + lax.axis_index("c")
        pltpu.sync_copy(data_hbm.at[pl.ds(wid * IPW, IPW)], idx_v)
        bufs = tuple(zip(rows_bufs, sems))

        def gather_off(s, b):
            return pl.multiple_of(s * L - _SHIFT[b], 8)

        def start(s, b):
            rows, sem = bufs[b]
            off = gather_off(s, b)
            pltpu.async_copy(
                table_hbm.at[idx_v.at[pl.ds(off, _GLEN[b])]], rows, sem)

        def wait(s, b):
            rows, sem = bufs[b]
            off = gather_off(s, b)
            pltpu.make_async_copy(
                table_hbm.at[idx_v.at[pl.ds(off, _GLEN[b])]], rows, sem
            ).wait()

        for b in range(NB):
            start(b, b)

        def seg_sum(rows, base, out_row):
            def body(r, accs):
                return tuple(
                    accs[d] + rows[base + r, pl.ds(d * 16, 16)]
                    for d in range(DV)
                )
            accs = lax.fori_loop(
                0, L, body,
                tuple(jnp.zeros((16,), jnp.float32) for _ in range(DV)),
            )
            for d in range(DV):
                acc[out_row, pl.ds(d * 16, 16)] = accs[d]

        def seg_round(i, carry):
            for b in range(NB):
                s = i * NB + b
                wait(s, b)
                seg_sum(bufs[b][0], _SHIFT[b], s)

                @pl.when(s + NB < SPW)
                def _():
                    start(s + NB, b)
            return carry

        lax.fori_loop(0, SPW // NB, seg_round, 0)
        for b in range(_TAIL):
            s = _FULL + b
            wait(s, b)
            seg_sum(bufs[b][0], _SHIFT[b], s)
        pltpu.sync_copy(acc, out_hbm.at[pl.ds(wid * SPW, SPW)])

    return k(data, table)


def _tc_mlp(d2, sums, t0, W1, b1, g1, be1, W2, b2, g2, be2, W3, b3, g3, be3,
            W4, b4):
    f32 = jnp.float32

    def body(d_ref, s_ref, t0_ref, w1, b1r, g1r, be1r, w2, b2r, g2r, be2r,
             w3, b3r, g3r, be3r, w4, b4r, o_ref):
        z = jnp.sum((d_ref[...] == 0).astype(f32), axis=1, keepdims=True)
        cnt = jnp.maximum(f32(L) - z, 1.0)
        pooled = (s_ref[...] - z * t0_ref[...]) / cnt
        inv = 1.0 / jnp.sqrt(f32(1.0 + 1e-5))
        h = jnp.dot(pooled, w1[...], preferred_element_type=f32) + b1r[...]
        h = jnp.maximum(h * inv * g1r[...] + be1r[...], 0.0)
        h = jnp.dot(h, w2[...], preferred_element_type=f32) + b2r[...]
        h = jnp.maximum(h * inv * g2r[...] + be2r[...], 0.0)
        h = jnp.dot(h, w3[...], preferred_element_type=f32) + b3r[...]
        h = jnp.maximum(h * inv * g3r[...] + be3r[...], 0.0)
        o_ref[...] = jnp.dot(h, w4[...], preferred_element_type=f32) + b4r[...]

    return pl.pallas_call(
        body,
        out_shape=jax.ShapeDtypeStruct((B, 2), f32),
    )(d2, sums, t0, W1, b1, g1, be1, W2, b2, g2, be2, W3, b3, g3, be3, W4, b4)


def kernel(data, offsets, table, W1, b1, g1, be1, W2, b2, g2, be2, W3, b3,
           g3, be3, W4, b4):
    del offsets
    sums = _sc_segment_sums(data, table)
    d2 = data.reshape(B, L)
    t0 = lax.slice(table, (0, 0), (1, D))
    r = lambda v: v.reshape(1, -1)
    return _tc_mlp(
        d2, sums, t0,
        W1, r(b1), r(g1), r(be1),
        W2, r(b2), r(g2), r(be2),
        W3, r(b3), r(g3), r(be3),
        W4, r(b4),
    )


# speedup vs baseline: 1.0896x; 1.0896x over previous
import functools

import jax
import jax.numpy as jnp
from jax import lax
from jax.experimental import pallas as pl
from jax.experimental.pallas import tpu as pltpu
from jax.experimental.pallas import tpu_sc as plsc

B = 4096
L = 50
D = 128
NC = 2
NS = 16
NW = NC * NS
SPW = B // NW
IPW = SPW * L
NB = 8
DV = D // 16
LP = 56
IRAW = IPW + 16
IPAD = SPW * LP + 16


def _sc_segment_sums(data, table):
    mesh = plsc.VectorSubcoreMesh(core_axis_name="c", subcore_axis_name="s")

    @functools.partial(
        pl.kernel,
        mesh=mesh,
        out_type=jax.ShapeDtypeStruct((B, D), jnp.float32),
        scratch_types=[
            pltpu.VMEM((IRAW,), jnp.int32),
            pltpu.VMEM((IPAD,), jnp.int32),
        ] + [
            pltpu.VMEM((L, D), jnp.float32) for _ in range(NB)
        ] + [
            pltpu.VMEM((SPW, D), jnp.float32),
        ] + [pltpu.SemaphoreType.DMA] * NB,
    )
    def k(data_hbm, table_hbm, out_hbm, idx_raw, idx_pad, *sc):
        rows_bufs, acc, sems = sc[:NB], sc[NB], sc[NB + 1:]
        wid = lax.axis_index("s") * NC + lax.axis_index("c")
        pltpu.sync_copy(data_hbm.at[pl.ds(wid * IPW, IPW)],
                        idx_raw.at[pl.ds(0, IPW)])
        bufs = tuple(zip(rows_bufs, sems))

        def repack(s):
            for j in range(4):
                idx_pad[pl.ds(s * LP + 16 * j, 16)] = (
                    idx_raw[pl.ds(s * L + 16 * j, 16)])

        def start(s, b):
            rows, sem = bufs[b]
            off = pl.multiple_of(s * LP, 8)
            pltpu.async_copy(
                table_hbm.at[idx_pad.at[pl.ds(off, L)]], rows, sem)

        def wait(s, b):
            rows, sem = bufs[b]
            off = pl.multiple_of(s * LP, 8)
            pltpu.make_async_copy(
                table_hbm.at[idx_pad.at[pl.ds(off, L)]], rows, sem
            ).wait()

        for b in range(NB):
            repack(b)
            start(b, b)

        def seg_sum(rows, base, out_row):
            def body(r, accs):
                return tuple(
                    accs[d] + rows[base + r, pl.ds(d * 16, 16)]
                    for d in range(DV)
                )
            accs = lax.fori_loop(
                0, L, body,
                tuple(jnp.zeros((16,), jnp.float32) for _ in range(DV)),
            )
            for d in range(DV):
                acc[out_row, pl.ds(d * 16, 16)] = accs[d]

        def seg_round(i, carry):
            for b in range(NB):
                s = i * NB + b
                wait(s, b)
                seg_sum(bufs[b][0], 0, s)

                @pl.when(s + NB < SPW)
                def _():
                    repack(s + NB)
                    start(s + NB, b)
            return carry

        lax.fori_loop(0, SPW // NB, seg_round, 0)
        pltpu.sync_copy(acc, out_hbm.at[pl.ds(wid * SPW, SPW)])

    return k(data, table)


def _tc_mlp(d2, sums, t0, W1, b1, g1, be1, W2, b2, g2, be2, W3, b3, g3, be3,
            W4, b4):
    f32 = jnp.float32

    def body(d_ref, s_ref, t0_ref, w1, b1r, g1r, be1r, w2, b2r, g2r, be2r,
             w3, b3r, g3r, be3r, w4, b4r, o_ref):
        z = jnp.sum((d_ref[...] == 0).astype(f32), axis=1, keepdims=True)
        cnt = jnp.maximum(f32(L) - z, 1.0)
        pooled = (s_ref[...] - z * t0_ref[...]) / cnt
        inv = 1.0 / jnp.sqrt(f32(1.0 + 1e-5))
        h = jnp.dot(pooled, w1[...], preferred_element_type=f32) + b1r[...]
        h = jnp.maximum(h * inv * g1r[...] + be1r[...], 0.0)
        h = jnp.dot(h, w2[...], preferred_element_type=f32) + b2r[...]
        h = jnp.maximum(h * inv * g2r[...] + be2r[...], 0.0)
        h = jnp.dot(h, w3[...], preferred_element_type=f32) + b3r[...]
        h = jnp.maximum(h * inv * g3r[...] + be3r[...], 0.0)
        o_ref[...] = jnp.dot(h, w4[...], preferred_element_type=f32) + b4r[...]

    return pl.pallas_call(
        body,
        out_shape=jax.ShapeDtypeStruct((B, 2), f32),
    )(d2, sums, t0, W1, b1, g1, be1, W2, b2, g2, be2, W3, b3, g3, be3, W4, b4)


def kernel(data, offsets, table, W1, b1, g1, be1, W2, b2, g2, be2, W3, b3,
           g3, be3, W4, b4):
    del offsets
    sums = _sc_segment_sums(data, table)
    d2 = data.reshape(B, L)
    t0 = lax.slice(table, (0, 0), (1, D))
    r = lambda v: v.reshape(1, -1)
    return _tc_mlp(
        d2, sums, t0,
        W1, r(b1), r(g1), r(be1),
        W2, r(b2), r(g2), r(be2),
        W3, r(b3), r(g3), r(be3),
        W4, r(b4),
    )


# speedup vs baseline: 1.0916x; 1.0018x over previous
import functools

import jax
import jax.numpy as jnp
from jax import lax
from jax.experimental import pallas as pl
from jax.experimental.pallas import tpu as pltpu
from jax.experimental.pallas import tpu_sc as plsc

B = 4096
L = 50
D = 128
NC = 2
NS = 16
NW = NC * NS
SPW = B // NW
IPW = SPW * L
NB = 8
DV = D // 16
LP = 56
IRAW = IPW + 16
IPAD = SPW * LP + 16


def _sc_segment_sums(data, table):
    mesh = plsc.VectorSubcoreMesh(core_axis_name="c", subcore_axis_name="s")

    @functools.partial(
        pl.kernel,
        mesh=mesh,
        out_type=jax.ShapeDtypeStruct((B, D), jnp.float32),
        scratch_types=[
            pltpu.VMEM((IRAW,), jnp.int32),
            pltpu.VMEM((IPAD,), jnp.int32),
        ] + [
            pltpu.VMEM((L, D), jnp.float32) for _ in range(NB)
        ] + [
            pltpu.VMEM((SPW, D), jnp.float32),
        ] + [pltpu.SemaphoreType.DMA] * NB,
    )
    def k(data_hbm, table_hbm, out_hbm, idx_raw, idx_pad, *sc):
        rows_bufs, acc, sems = sc[:NB], sc[NB], sc[NB + 1:]
        wid = lax.axis_index("s") * NC + lax.axis_index("c")
        pltpu.sync_copy(data_hbm.at[pl.ds(wid * IPW, IPW)],
                        idx_raw.at[pl.ds(0, IPW)])
        bufs = tuple(zip(rows_bufs, sems))

        def repack(s):
            for j in range(4):
                idx_pad[pl.ds(s * LP + 16 * j, 16)] = (
                    idx_raw[pl.ds(s * L + 16 * j, 16)])

        def start(s, b):
            rows, sem = bufs[b]
            off = pl.multiple_of(s * LP, 8)
            pltpu.async_copy(
                table_hbm.at[idx_pad.at[pl.ds(off, L)]], rows, sem)

        def wait(s, b):
            rows, sem = bufs[b]
            off = pl.multiple_of(s * LP, 8)
            pltpu.make_async_copy(
                table_hbm.at[idx_pad.at[pl.ds(off, L)]], rows, sem
            ).wait()

        for b in range(NB):
            repack(b)
            start(b, b)

        def seg_sum(rows, base, out_row):
            def body(r, accs):
                return tuple(
                    accs[d] + rows[base + r, pl.ds(d * 16, 16)]
                    for d in range(DV)
                )
            accs = lax.fori_loop(
                0, L, body,
                tuple(jnp.zeros((16,), jnp.float32) for _ in range(DV)),
            )
            for d in range(DV):
                acc[out_row, pl.ds(d * 16, 16)] = accs[d]

        def seg_round(i, carry):
            for b in range(NB):
                s = i * NB + b
                wait(s, b)
                seg_sum(bufs[b][0], 0, s)

                @pl.when(s + NB < SPW)
                def _():
                    repack(s + NB)
                    start(s + NB, b)
            return carry

        lax.fori_loop(0, SPW // NB, seg_round, 0)
        pltpu.sync_copy(acc, out_hbm.at[pl.ds(wid * SPW, SPW)])

    return k(data, table)


def _tc_mlp(d2, sums, t0, W1, b1, g1, be1, W2, b2, g2, be2, W3, b3, g3, be3,
            W4, b4):
    f32 = jnp.float32

    def body(d_ref, s_ref, t0_ref, w1, b1r, g1r, be1r, w2, b2r, g2r, be2r,
             w3, b3r, g3r, be3r, w4, b4r, o_ref):
        z = jnp.sum((d_ref[...] == 0).astype(f32), axis=1, keepdims=True)
        cnt = jnp.maximum(f32(L) - z, 1.0)
        pooled = (s_ref[...] - z * t0_ref[...]) / cnt
        inv = 1.0 / jnp.sqrt(f32(1.0 + 1e-5))
        h = jnp.dot(pooled, w1[...], preferred_element_type=f32) + b1r[...]
        h = jnp.maximum(h * inv * g1r[...] + be1r[...], 0.0)
        h = jnp.dot(h, w2[...], preferred_element_type=f32) + b2r[...]
        h = jnp.maximum(h * inv * g2r[...] + be2r[...], 0.0)
        h = jnp.dot(h, w3[...], preferred_element_type=f32) + b3r[...]
        h = jnp.maximum(h * inv * g3r[...] + be3r[...], 0.0)
        o_ref[...] = jnp.dot(h, w4[...], preferred_element_type=f32) + b4r[...]

    return pl.pallas_call(
        body,
        out_shape=jax.ShapeDtypeStruct((B, D), f32),
    )(d2, sums, t0, W1, b1, g1, be1, W2, b2, g2, be2, W3, b3, g3, be3, W4, b4)


def kernel(data, offsets, table, W1, b1, g1, be1, W2, b2, g2, be2, W3, b3,
           g3, be3, W4, b4):
    del offsets
    sums = _sc_segment_sums(data, table)
    d2 = data.reshape(B, L)
    t0 = lax.slice(table, (0, 0), (1, D))
    r = lambda v: v.reshape(1, -1)
    W4p = jnp.zeros((64, D), W4.dtype).at[:, :2].set(W4)
    b4p = jnp.zeros((1, D), b4.dtype).at[:, :2].set(r(b4))
    wide = _tc_mlp(
        d2, sums, t0,
        W1, r(b1), r(g1), r(be1),
        W2, r(b2), r(g2), r(be2),
        W3, r(b3), r(g3), r(be3),
        W4p, b4p,
    )
    return lax.slice(wide, (0, 0), (B, 2))


# speedup vs baseline: 1.0919x; 1.0004x over previous
import functools

import jax
import jax.numpy as jnp
from jax import lax
from jax.experimental import pallas as pl
from jax.experimental.pallas import tpu as pltpu
from jax.experimental.pallas import tpu_sc as plsc

B = 4096
L = 50
D = 128
NC = 2
NS = 16
NW = NC * NS
SPW = B // NW
IPW = SPW * L
NB = 8
DV = D // 16
LP = 56
IRAW = IPW + 16
IPAD = SPW * LP + 16


def _sc_segment_sums(data, table):
    mesh = plsc.VectorSubcoreMesh(core_axis_name="c", subcore_axis_name="s")

    @functools.partial(
        pl.kernel,
        mesh=mesh,
        out_type=jax.ShapeDtypeStruct((B, D), jnp.float32),
        scratch_types=[
            pltpu.VMEM((IRAW,), jnp.int32),
            pltpu.VMEM((IPAD,), jnp.int32),
        ] + [
            pltpu.VMEM((L, D), jnp.float32) for _ in range(NB)
        ] + [
            pltpu.VMEM((SPW, D), jnp.float32),
        ] + [pltpu.SemaphoreType.DMA] * NB,
    )
    def k(data_hbm, table_hbm, out_hbm, idx_raw, idx_pad, *sc):
        rows_bufs, acc, sems = sc[:NB], sc[NB], sc[NB + 1:]
        wid = lax.axis_index("s") * NC + lax.axis_index("c")
        pltpu.sync_copy(data_hbm.at[pl.ds(wid * IPW, IPW)],
                        idx_raw.at[pl.ds(0, IPW)])
        bufs = tuple(zip(rows_bufs, sems))

        def repack(s):
            for j in range(4):
                idx_pad[pl.ds(s * LP + 16 * j, 16)] = (
                    idx_raw[pl.ds(s * L + 16 * j, 16)])

        def start(s, b):
            rows, sem = bufs[b]
            off = pl.multiple_of(s * LP, 8)
            pltpu.async_copy(
                table_hbm.at[idx_pad.at[pl.ds(off, L)]], rows, sem)

        def wait(s, b):
            rows, sem = bufs[b]
            off = pl.multiple_of(s * LP, 8)
            pltpu.make_async_copy(
                table_hbm.at[idx_pad.at[pl.ds(off, L)]], rows, sem
            ).wait()

        for b in range(NB):
            repack(b)
            start(b, b)

        def seg_sum(rows, base, out_row):
            def body(r, accs):
                return tuple(
                    accs[d] + rows[base + r, pl.ds(d * 16, 16)]
                    for d in range(DV)
                )
            accs = lax.fori_loop(
                0, L, body,
                tuple(jnp.zeros((16,), jnp.float32) for _ in range(DV)),
            )
            for d in range(DV):
                acc[out_row, pl.ds(d * 16, 16)] = accs[d]

        def seg_round(i, carry):
            for b in range(NB):
                s = i * NB + b
                wait(s, b)
                seg_sum(bufs[b][0], 0, s)

                @pl.when(s + NB < SPW)
                def _():
                    repack(s + NB)
                    start(s + NB, b)
            return carry

        lax.fori_loop(0, SPW // NB, seg_round, 0)
        pltpu.sync_copy(acc, out_hbm.at[pl.ds(wid * SPW, SPW)])

    return k(data, table)


def _tc_mlp(d2, sums, table, W1, b1, g1, be1, W2, b2, g2, be2, W3, b3, g3,
            be3, W4, b4):
    f32 = jnp.float32

    def body(d_ref, s_ref, t0_ref, w1, b1r, g1r, be1r, w2, b2r, g2r, be2r,
             w3, b3r, g3r, be3r, w4, b4r, o_ref):
        z = jnp.sum((d_ref[...] == 0).astype(f32), axis=1, keepdims=True)
        cnt = jnp.maximum(f32(L) - z, 1.0)
        pooled = (s_ref[...] - z * t0_ref[0:1, :]) / cnt
        inv = 1.0 / jnp.sqrt(f32(1.0 + 1e-5))
        h = jnp.dot(pooled, w1[...], preferred_element_type=f32) + b1r[...]
        h = jnp.maximum(h * inv * g1r[...] + be1r[...], 0.0)
        h = jnp.dot(h, w2[...], preferred_element_type=f32) + b2r[...]
        h = jnp.maximum(h * inv * g2r[...] + be2r[...], 0.0)
        h = jnp.dot(h, w3[...], preferred_element_type=f32) + b3r[...]
        h = jnp.maximum(h * inv * g3r[...] + be3r[...], 0.0)
        o_ref[...] = jnp.dot(h, w4[...], preferred_element_type=f32) + b4r[...]

    whole = lambda a: pl.BlockSpec(a.shape, lambda i: (0,) * a.ndim)
    args = (d2, sums, table, W1, b1, g1, be1, W2, b2, g2, be2, W3, b3, g3,
            be3, W4, b4)
    specs = [whole(a) for a in args]
    specs[2] = pl.BlockSpec((8, D), lambda i: (0, 0))
    return pl.pallas_call(
        body,
        grid=(1,),
        in_specs=specs,
        out_specs=pl.BlockSpec((B, D), lambda i: (0, 0)),
        out_shape=jax.ShapeDtypeStruct((B, D), f32),
    )(*args)


def kernel(data, offsets, table, W1, b1, g1, be1, W2, b2, g2, be2, W3, b3,
           g3, be3, W4, b4):
    del offsets
    sums = _sc_segment_sums(data, table)
    d2 = data.reshape(B, L)
    r = lambda v: v.reshape(1, -1)
    W4p = jnp.zeros((64, D), W4.dtype).at[:, :2].set(W4)
    b4p = jnp.zeros((1, D), b4.dtype).at[:, :2].set(r(b4))
    wide = _tc_mlp(
        d2, sums, table,
        W1, r(b1), r(g1), r(be1),
        W2, r(b2), r(g2), r(be2),
        W3, r(b3), r(g3), r(be3),
        W4p, b4p,
    )
    return lax.slice(wide, (0, 0), (B, 2))
